# baseline (speedup 1.0000x reference)
import functools

import jax
import jax.numpy as jnp
from jax import lax
from jax.experimental import pallas as pl
from jax.experimental.pallas import tpu as pltpu
from jax.experimental.pallas import tpu_sc as plsc

N_NODES = 10000
N_EDGES = 320000
IN_FEATS = 10000
OUT_FEATS = 128

NC = 2
NS = 16
L = 16
FH = OUT_FEATS // NC

NPT = 640
N_PAD = NS * NPT
NODE_CHUNKS = NPT // 128

SUB = 256
NSUB = 80
PAIRS = NSUB // 2
EPT = NSUB * SUB
E_PAD = NS * EPT


def _rsqrt_inplace(ref, n_vecs):
    def body(i, carry):
        x = jnp.maximum(ref[pl.ds(i * L, L)], 1.0)
        bits = lax.bitcast_convert_type(x, jnp.int32)
        y = lax.bitcast_convert_type(
            jnp.int32(0x5F3759DF) - lax.shift_right_arithmetic(bits, 1),
            jnp.float32)
        for _ in range(3):
            y = y * (1.5 - 0.5 * x * y * y)
        ref[pl.ds(i * L, L)] = y
        return carry

    lax.fori_loop(0, n_vecs, body, 0)


def _sc_body(feat_ref, src_ref, dst_ref, emb2_ref, bias_ref, out_ref,
             x_sp, accum, outdeg, indeg,
             ssub_a, dsub_a, ssub_b, dsub_b, featbuf, norm_v,
             rows_a, rows_b, rows_c, ones_v, biasv,
             gsem_a, gsem_b):
    c = lax.axis_index("c")
    s = lax.axis_index("s")
    base_n = s * NPT
    rows0 = rows_c

    zeros16 = jnp.zeros((L,), jnp.float32)

    def zero_rows(r, carry):
        for f in range(FH // L):
            rows0[r, pl.ds(f * L, L)] = zeros16
        return carry

    lax.fori_loop(0, 128, zero_rows, 0)

    def zero_norm(i, carry):
        norm_v[pl.ds(i * L, L)] = zeros16
        return carry

    lax.fori_loop(0, NPT // L, zero_norm, 0)

    ones16 = jnp.ones((L,), jnp.float32)

    def fill_ones(k, carry):
        ones_v[pl.ds(k * L, L)] = ones16
        return carry

    lax.fori_loop(0, SUB // L, fill_ones, 0)

    for j in range(NODE_CHUNKS):
        pltpu.sync_copy(rows0, accum.at[pl.ds(base_n + j * 128, 128)])
    pltpu.sync_copy(norm_v, outdeg.at[pl.ds(base_n, NPT)])
    pltpu.sync_copy(norm_v, indeg.at[pl.ds(base_n, NPT)])

    pltpu.sync_copy(feat_ref.at[pl.ds(base_n, NPT)], featbuf)
    pltpu.sync_copy(bias_ref.at[pl.ds(c * FH, FH)], biasv)

    coff = (c * IN_FEATS).astype(jnp.int32)

    def add_off(i, carry):
        featbuf[pl.ds(i * L, L)] = featbuf[pl.ds(i * L, L)] + coff
        return carry

    lax.fori_loop(0, NPT // L, add_off, 0)

    plsc.subcore_barrier()

    def hist_sub(u, carry):
        pltpu.sync_copy(src_ref.at[s, pl.ds(u * SUB, SUB)], ssub_a)
        pltpu.sync_copy(dst_ref.at[s, pl.ds(u * SUB, SUB)], dsub_a)
        pltpu.sync_copy(ones_v, outdeg.at[ssub_a], add=True)
        pltpu.sync_copy(ones_v, indeg.at[dsub_a], add=True)
        return carry

    lax.fori_loop(0, NSUB, hist_sub, 0)
    plsc.subcore_barrier()

    pltpu.sync_copy(outdeg.at[pl.ds(base_n, NPT)], norm_v)
    _rsqrt_inplace(norm_v, NPT // L)

    lane_iota = lax.iota(jnp.int32, L)

    def scale_rows(j, bias_vecs=None):
        def group(g, carry2):
            nv16 = norm_v[pl.ds(j * 128 + g * L, L)]
            for r16 in range(L):
                bc = jnp.full((L,), jnp.sum(jnp.where(lane_iota == r16,
                                                      nv16, 0.0)))
                r = g * L + r16
                for f in range(FH // L):
                    v = rows0[r, pl.ds(f * L, L)] * bc
                    if bias_vecs is not None:
                        v = v + bias_vecs[f]
                    rows0[r, pl.ds(f * L, L)] = v
            return carry2

        lax.fori_loop(0, 128 // L, group, 0)

    def build_chunk(j, carry):
        pltpu.sync_copy(emb2_ref.at[featbuf.at[pl.ds(j * 128, 128)]], rows0)
        scale_rows(j)
        pltpu.sync_copy(rows0,
                        x_sp.at[pl.ds(c * N_PAD + base_n + j * 128, 128)])
        return carry

    lax.fori_loop(0, NODE_CHUNKS, build_chunk, 0)

    pltpu.sync_copy(indeg.at[pl.ds(base_n, NPT)], norm_v)
    _rsqrt_inplace(norm_v, NPT // L)
    plsc.subcore_barrier()

    xoff = (c * N_PAD).astype(jnp.int32)

    def edge_pair(t, carry):
        pltpu.sync_copy(src_ref.at[s, pl.ds((2 * t) * SUB, SUB)], ssub_a)
        pltpu.sync_copy(src_ref.at[s, pl.ds((2 * t + 1) * SUB, SUB)], ssub_b)
        for i in range(SUB // L):
            ssub_a[pl.ds(i * L, L)] = ssub_a[pl.ds(i * L, L)] + xoff
            ssub_b[pl.ds(i * L, L)] = ssub_b[pl.ds(i * L, L)] + xoff
        da = pltpu.async_copy(x_sp.at[ssub_a], rows_a, gsem_a)
        db = pltpu.async_copy(x_sp.at[ssub_b], rows_b, gsem_b)
        pltpu.sync_copy(dst_ref.at[s, pl.ds((2 * t) * SUB, SUB)], dsub_a)
        pltpu.sync_copy(dst_ref.at[s, pl.ds((2 * t + 1) * SUB, SUB)], dsub_b)
        da.wait()
        pltpu.sync_copy(rows_a, accum.at[dsub_a], add=True)
        db.wait()
        pltpu.sync_copy(rows_b, accum.at[dsub_b], add=True)
        return carry

    lax.fori_loop(0, PAIRS, edge_pair, 0)
    plsc.subcore_barrier()

    bias_vecs = [biasv[pl.ds(f * L, L)] for f in range(FH // L)]

    def out_chunk(j, carry):
        pltpu.sync_copy(accum.at[pl.ds(base_n + j * 128, 128)], rows0)
        scale_rows(j, bias_vecs)
        pltpu.sync_copy(rows0,
                        out_ref.at[c, pl.ds(base_n + j * 128, 128)])
        return carry

    lax.fori_loop(0, NODE_CHUNKS, out_chunk, 0)


@functools.partial(jax.jit, static_argnames=())
def kernel(feat, edge_index, emb, bias):
    feat = feat.astype(jnp.int32)
    src = edge_index[0].astype(jnp.int32)
    dst = edge_index[1].astype(jnp.int32)

    emb2 = jnp.concatenate([emb[:, :FH], emb[:, FH:]], axis=0)

    feat_p = jnp.concatenate(
        [feat, jnp.zeros((N_PAD - N_NODES,), jnp.int32)])
    npad = E_PAD - N_EDGES
    pad_idx = (jnp.arange(npad, dtype=jnp.int32) % (N_PAD - N_NODES)
               ) + N_NODES
    src_p = jnp.concatenate([src, pad_idx]).reshape(NS, EPT)
    dst_p = jnp.concatenate([dst, pad_idx]).reshape(NS, EPT)

    mesh = plsc.VectorSubcoreMesh(core_axis_name="c", subcore_axis_name="s",
                                  num_cores=NC, num_subcores=NS)
    out = pl.kernel(
        _sc_body,
        out_type=jax.ShapeDtypeStruct((NC, N_PAD, FH), jnp.float32),
        mesh=mesh,
        compiler_params=pltpu.CompilerParams(needs_layout_passes=False,
                                             use_tc_tiling_on_sc=False),
        scratch_types=[
            pltpu.HBM((NC * N_PAD, FH), jnp.float32),
            pltpu.VMEM_SHARED((N_PAD, FH), jnp.float32),
            pltpu.VMEM_SHARED((N_PAD,), jnp.float32),
            pltpu.VMEM_SHARED((N_PAD,), jnp.float32),
            pltpu.VMEM((SUB,), jnp.int32),
            pltpu.VMEM((SUB,), jnp.int32),
            pltpu.VMEM((SUB,), jnp.int32),
            pltpu.VMEM((SUB,), jnp.int32),
            pltpu.VMEM((NPT,), jnp.int32),
            pltpu.VMEM((NPT,), jnp.float32),
            pltpu.VMEM((SUB, FH), jnp.float32),
            pltpu.VMEM((SUB, FH), jnp.float32),
            pltpu.VMEM((128, FH), jnp.float32),
            pltpu.VMEM((SUB,), jnp.float32),
            pltpu.VMEM((FH,), jnp.float32),
            pltpu.SemaphoreType.DMA,
            pltpu.SemaphoreType.DMA,
        ],
    )(feat_p, src_p, dst_p, emb2, bias)
    return jnp.concatenate([out[0, :N_NODES], out[1, :N_NODES]], axis=1)


# speedup vs baseline: 1.3034x; 1.3034x over previous
import functools

import jax
import jax.numpy as jnp
from jax import lax
from jax.experimental import pallas as pl
from jax.experimental.pallas import tpu as pltpu
from jax.experimental.pallas import tpu_sc as plsc

N_NODES = 10000
N_EDGES = 320000
IN_FEATS = 10000
OUT_FEATS = 128

NC = 2
NS = 16
L = 16
FH = OUT_FEATS // NC

NPT = 640
N_PAD = NS * NPT
NODE_CHUNKS = NPT // 128

SUB = 200
NSUB = 100
PAIRS = NSUB // 2
HSUB = 1000
HPAIRS = 10
EPT = NSUB * SUB


def _rsqrt_inplace(ref, n_vecs):
    def body(i, carry):
        x = jnp.maximum(ref[pl.ds(i * L, L)], 1.0)
        bits = lax.bitcast_convert_type(x, jnp.int32)
        y = lax.bitcast_convert_type(
            jnp.int32(0x5F3759DF) - lax.shift_right_arithmetic(bits, 1),
            jnp.float32)
        for _ in range(3):
            y = y * (1.5 - 0.5 * x * y * y)
        ref[pl.ds(i * L, L)] = y
        return carry

    lax.fori_loop(0, n_vecs, body, 0)


def _sc_body(feat_ref, src_ref, dst_ref, emb2_ref, bias_ref, out_ref,
             x_sp, accum, outdeg, indeg,
             ssub_a, dsub_a, ssub_b, dsub_b, hs_a, hd_a, hs_b, hd_b,
             featbuf, norm_v, rows_a, rows_b, rows_c, ones_v, biasv,
             gsem_a, gsem_b, hsem_a, hsem_b):
    c = lax.axis_index("c")
    s = lax.axis_index("s")
    base_n = s * NPT
    rows0 = rows_c

    zeros16 = jnp.zeros((L,), jnp.float32)

    def zero_rows(r, carry):
        for f in range(FH // L):
            rows0[r, pl.ds(f * L, L)] = zeros16
        return carry

    lax.fori_loop(0, 128, zero_rows, 0)

    def zero_norm(i, carry):
        norm_v[pl.ds(i * L, L)] = zeros16
        return carry

    lax.fori_loop(0, NPT // L, zero_norm, 0)

    ones16 = jnp.ones((L,), jnp.float32)

    def fill_ones(k, carry):
        ones_v[pl.ds(k * L, L)] = ones16
        return carry

    lax.fori_loop(0, HSUB // L, fill_ones, 0)

    for j in range(NODE_CHUNKS):
        pltpu.sync_copy(rows0, accum.at[pl.ds(base_n + j * 128, 128)])
    pltpu.sync_copy(norm_v, outdeg.at[pl.ds(base_n, NPT)])
    pltpu.sync_copy(norm_v, indeg.at[pl.ds(base_n, NPT)])

    pltpu.sync_copy(feat_ref.at[pl.ds(base_n, NPT)], featbuf)
    pltpu.sync_copy(bias_ref.at[pl.ds(c * FH, FH)], biasv)

    coff = (c * IN_FEATS).astype(jnp.int32)

    def add_off(i, carry):
        featbuf[pl.ds(i * L, L)] = featbuf[pl.ds(i * L, L)] + coff
        return carry

    lax.fori_loop(0, NPT // L, add_off, 0)

    plsc.subcore_barrier()

    def hist_pair(u, carry):
        pltpu.sync_copy(src_ref.at[s, pl.ds((2 * u) * HSUB, HSUB)], hs_a)
        pltpu.sync_copy(dst_ref.at[s, pl.ds((2 * u) * HSUB, HSUB)], hd_a)
        da = pltpu.async_copy(ones_v, outdeg.at[hs_a], hsem_a, add=True)
        db = pltpu.async_copy(ones_v, indeg.at[hd_a], hsem_b, add=True)
        pltpu.sync_copy(src_ref.at[s, pl.ds((2 * u + 1) * HSUB, HSUB)], hs_b)
        pltpu.sync_copy(dst_ref.at[s, pl.ds((2 * u + 1) * HSUB, HSUB)], hd_b)
        da.wait()
        db.wait()
        dc = pltpu.async_copy(ones_v, outdeg.at[hs_b], hsem_a, add=True)
        dd = pltpu.async_copy(ones_v, indeg.at[hd_b], hsem_b, add=True)
        dc.wait()
        dd.wait()
        return carry

    lax.fori_loop(0, HPAIRS, hist_pair, 0)
    plsc.subcore_barrier()

    pltpu.sync_copy(outdeg.at[pl.ds(base_n, NPT)], norm_v)
    _rsqrt_inplace(norm_v, NPT // L)

    lane_iota = lax.iota(jnp.int32, L)

    def scale_rows(j, bias_vecs=None):
        def group(g, carry2):
            nv16 = norm_v[pl.ds(j * 128 + g * L, L)]
            for r16 in range(L):
                bc = jnp.full((L,), jnp.sum(jnp.where(lane_iota == r16,
                                                      nv16, 0.0)))
                r = g * L + r16
                for f in range(FH // L):
                    v = rows0[r, pl.ds(f * L, L)] * bc
                    if bias_vecs is not None:
                        v = v + bias_vecs[f]
                    rows0[r, pl.ds(f * L, L)] = v
            return carry2

        lax.fori_loop(0, 128 // L, group, 0)

    def build_chunk(j, carry):
        pltpu.sync_copy(emb2_ref.at[featbuf.at[pl.ds(j * 128, 128)]], rows0)
        scale_rows(j)
        pltpu.sync_copy(rows0, x_sp.at[pl.ds(base_n + j * 128, 128)])
        return carry

    lax.fori_loop(0, NODE_CHUNKS, build_chunk, 0)

    pltpu.sync_copy(indeg.at[pl.ds(base_n, NPT)], norm_v)
    _rsqrt_inplace(norm_v, NPT // L)
    plsc.subcore_barrier()

    def edge_pair(t, carry):
        pltpu.sync_copy(src_ref.at[s, pl.ds((2 * t) * SUB, SUB)], ssub_a)
        pltpu.sync_copy(src_ref.at[s, pl.ds((2 * t + 1) * SUB, SUB)], ssub_b)
        da = pltpu.async_copy(x_sp.at[ssub_a], rows_a, gsem_a)
        db = pltpu.async_copy(x_sp.at[ssub_b], rows_b, gsem_b)
        pltpu.sync_copy(dst_ref.at[s, pl.ds((2 * t) * SUB, SUB)], dsub_a)
        pltpu.sync_copy(dst_ref.at[s, pl.ds((2 * t + 1) * SUB, SUB)], dsub_b)
        da.wait()
        pltpu.sync_copy(rows_a, accum.at[dsub_a], add=True)
        db.wait()
        pltpu.sync_copy(rows_b, accum.at[dsub_b], add=True)
        return carry

    lax.fori_loop(0, PAIRS, edge_pair, 0)
    plsc.subcore_barrier()

    bias_vecs = [biasv[pl.ds(f * L, L)] for f in range(FH // L)]

    def out_chunk(j, carry):
        pltpu.sync_copy(accum.at[pl.ds(base_n + j * 128, 128)], rows0)
        scale_rows(j, bias_vecs)
        pltpu.sync_copy(rows0,
                        out_ref.at[c, pl.ds(base_n + j * 128, 128)])
        return carry

    lax.fori_loop(0, NODE_CHUNKS, out_chunk, 0)


@functools.partial(jax.jit, static_argnames=())
def kernel(feat, edge_index, emb, bias):
    feat = feat.astype(jnp.int32)
    src = edge_index[0].astype(jnp.int32)
    dst = edge_index[1].astype(jnp.int32)

    emb2 = jnp.concatenate([emb[:, :FH], emb[:, FH:]], axis=0)

    feat_p = jnp.concatenate(
        [feat, jnp.zeros((N_PAD - N_NODES,), jnp.int32)])
    src_p = src.reshape(NS, EPT)
    dst_p = dst.reshape(NS, EPT)

    mesh = plsc.VectorSubcoreMesh(core_axis_name="c", subcore_axis_name="s",
                                  num_cores=NC, num_subcores=NS)
    out = pl.kernel(
        _sc_body,
        out_type=jax.ShapeDtypeStruct((NC, N_PAD, FH), jnp.float32),
        mesh=mesh,
        compiler_params=pltpu.CompilerParams(needs_layout_passes=False,
                                             use_tc_tiling_on_sc=False),
        scratch_types=[
            pltpu.VMEM_SHARED((N_PAD, FH), jnp.float32),
            pltpu.VMEM_SHARED((N_PAD, FH), jnp.float32),
            pltpu.VMEM_SHARED((N_PAD,), jnp.float32),
            pltpu.VMEM_SHARED((N_PAD,), jnp.float32),
            pltpu.VMEM((SUB,), jnp.int32),
            pltpu.VMEM((SUB,), jnp.int32),
            pltpu.VMEM((SUB,), jnp.int32),
            pltpu.VMEM((SUB,), jnp.int32),
            pltpu.VMEM((HSUB,), jnp.int32),
            pltpu.VMEM((HSUB,), jnp.int32),
            pltpu.VMEM((HSUB,), jnp.int32),
            pltpu.VMEM((HSUB,), jnp.int32),
            pltpu.VMEM((NPT,), jnp.int32),
            pltpu.VMEM((NPT,), jnp.float32),
            pltpu.VMEM((SUB, FH), jnp.float32),
            pltpu.VMEM((SUB, FH), jnp.float32),
            pltpu.VMEM((128, FH), jnp.float32),
            pltpu.VMEM((HSUB,), jnp.float32),
            pltpu.VMEM((FH,), jnp.float32),
            pltpu.SemaphoreType.DMA,
            pltpu.SemaphoreType.DMA,
            pltpu.SemaphoreType.DMA,
            pltpu.SemaphoreType.DMA,
        ],
    )(feat_p, src_p, dst_p, emb2, bias)
    return jnp.concatenate([out[0, :N_NODES], out[1, :N_NODES]], axis=1)


# speedup vs baseline: 1.4719x; 1.1293x over previous
import functools

import jax
import jax.numpy as jnp
from jax import lax
from jax.experimental import pallas as pl
from jax.experimental.pallas import tpu as pltpu
from jax.experimental.pallas import tpu_sc as plsc

N_NODES = 10000
N_EDGES = 320000
IN_FEATS = 10000
OUT_FEATS = 128

NC = 2
NS = 16
L = 16
FH = OUT_FEATS // NC

NPT = 640
N_PAD = NS * NPT
NODE_CHUNKS = NPT // 128

SUB = 400
NSUB = 50
HSUB = 2000
HPAIRS = 5
EPT = NSUB * SUB


def _rsqrt_inplace(ref, n_vecs):
    def body(i, carry):
        x = jnp.maximum(ref[pl.ds(i * L, L)], 1.0)
        bits = lax.bitcast_convert_type(x, jnp.int32)
        y = lax.bitcast_convert_type(
            jnp.int32(0x5F3759DF) - lax.shift_right_arithmetic(bits, 1),
            jnp.float32)
        for _ in range(3):
            y = y * (1.5 - 0.5 * x * y * y)
        ref[pl.ds(i * L, L)] = y
        return carry

    lax.fori_loop(0, n_vecs, body, 0)


def _sc_body(feat_ref, src_ref, dst_ref, emb2_ref, bias_ref, out_ref,
             x_sp, accum, outdeg, indeg,
             ssub_a, dsub_a, hs_a, hd_a, hs_b, hd_b,
             featbuf, norm_v, rows_a, rows_c, ones_v, biasv,
             gsem_a, hsem_a, hsem_b):
    c = lax.axis_index("c")
    s = lax.axis_index("s")
    base_n = s * NPT
    rows0 = rows_c

    zeros16 = jnp.zeros((L,), jnp.float32)

    def zero_rows(r, carry):
        for f in range(FH // L):
            rows0[r, pl.ds(f * L, L)] = zeros16
        return carry

    lax.fori_loop(0, 128, zero_rows, 0)

    def zero_norm(i, carry):
        norm_v[pl.ds(i * L, L)] = zeros16
        return carry

    lax.fori_loop(0, NPT // L, zero_norm, 0)

    ones16 = jnp.ones((L,), jnp.float32)

    def fill_ones(k, carry):
        ones_v[pl.ds(k * L, L)] = ones16
        return carry

    lax.fori_loop(0, HSUB // L, fill_ones, 0)

    for j in range(NODE_CHUNKS):
        pltpu.sync_copy(rows0, accum.at[pl.ds(base_n + j * 128, 128)])
    pltpu.sync_copy(norm_v, outdeg.at[pl.ds(base_n, NPT)])
    pltpu.sync_copy(norm_v, indeg.at[pl.ds(base_n, NPT)])

    pltpu.sync_copy(feat_ref.at[pl.ds(base_n, NPT)], featbuf)
    pltpu.sync_copy(bias_ref.at[pl.ds(c * FH, FH)], biasv)

    coff = (c * IN_FEATS).astype(jnp.int32)

    def add_off(i, carry):
        featbuf[pl.ds(i * L, L)] = featbuf[pl.ds(i * L, L)] + coff
        return carry

    lax.fori_loop(0, NPT // L, add_off, 0)

    plsc.subcore_barrier()

    def hist_pair(u, carry):
        pltpu.sync_copy(src_ref.at[s, pl.ds((2 * u) * HSUB, HSUB)], hs_a)
        pltpu.sync_copy(dst_ref.at[s, pl.ds((2 * u) * HSUB, HSUB)], hd_a)
        da = pltpu.async_copy(ones_v, outdeg.at[hs_a], hsem_a, add=True)
        db = pltpu.async_copy(ones_v, indeg.at[hd_a], hsem_b, add=True)
        pltpu.sync_copy(src_ref.at[s, pl.ds((2 * u + 1) * HSUB, HSUB)], hs_b)
        pltpu.sync_copy(dst_ref.at[s, pl.ds((2 * u + 1) * HSUB, HSUB)], hd_b)
        da.wait()
        db.wait()
        dc = pltpu.async_copy(ones_v, outdeg.at[hs_b], hsem_a, add=True)
        dd = pltpu.async_copy(ones_v, indeg.at[hd_b], hsem_b, add=True)
        dc.wait()
        dd.wait()
        return carry

    lax.fori_loop(0, HPAIRS, hist_pair, 0)
    plsc.subcore_barrier()

    pltpu.sync_copy(outdeg.at[pl.ds(base_n, NPT)], norm_v)
    _rsqrt_inplace(norm_v, NPT // L)

    lane_iota = lax.iota(jnp.int32, L)

    def scale_rows(j, bias_vecs=None):
        def group(g, carry2):
            nv16 = norm_v[pl.ds(j * 128 + g * L, L)]
            for r16 in range(L):
                bc = jnp.full((L,), jnp.sum(jnp.where(lane_iota == r16,
                                                      nv16, 0.0)))
                r = g * L + r16
                for f in range(FH // L):
                    v = rows0[r, pl.ds(f * L, L)] * bc
                    if bias_vecs is not None:
                        v = v + bias_vecs[f]
                    rows0[r, pl.ds(f * L, L)] = v
            return carry2

        lax.fori_loop(0, 128 // L, group, 0)

    def build_chunk(j, carry):
        pltpu.sync_copy(emb2_ref.at[featbuf.at[pl.ds(j * 128, 128)]], rows0)
        scale_rows(j)
        pltpu.sync_copy(rows0, x_sp.at[pl.ds(base_n + j * 128, 128)])
        return carry

    lax.fori_loop(0, NODE_CHUNKS, build_chunk, 0)

    pltpu.sync_copy(indeg.at[pl.ds(base_n, NPT)], norm_v)
    _rsqrt_inplace(norm_v, NPT // L)
    plsc.subcore_barrier()

    def edge_sub(t, carry):
        pltpu.sync_copy(src_ref.at[s, pl.ds(t * SUB, SUB)], ssub_a)
        da = pltpu.async_copy(x_sp.at[ssub_a], rows_a, gsem_a)
        pltpu.sync_copy(dst_ref.at[s, pl.ds(t * SUB, SUB)], dsub_a)
        da.wait()
        pltpu.sync_copy(rows_a, accum.at[dsub_a], add=True)
        return carry

    lax.fori_loop(0, NSUB, edge_sub, 0)
    plsc.subcore_barrier()

    bias_vecs = [biasv[pl.ds(f * L, L)] for f in range(FH // L)]

    def out_chunk(j, carry):
        pltpu.sync_copy(accum.at[pl.ds(base_n + j * 128, 128)], rows0)
        scale_rows(j, bias_vecs)
        pltpu.sync_copy(rows0,
                        out_ref.at[c, pl.ds(base_n + j * 128, 128)])
        return carry

    lax.fori_loop(0, NODE_CHUNKS, out_chunk, 0)


@functools.partial(jax.jit, static_argnames=())
def kernel(feat, edge_index, emb, bias):
    feat = feat.astype(jnp.int32)
    src = edge_index[0].astype(jnp.int32)
    dst = edge_index[1].astype(jnp.int32)

    emb2 = jnp.concatenate([emb[:, :FH], emb[:, FH:]], axis=0)

    feat_p = jnp.concatenate(
        [feat, jnp.zeros((N_PAD - N_NODES,), jnp.int32)])
    src_p = src.reshape(NS, EPT)
    dst_p = dst.reshape(NS, EPT)

    mesh = plsc.VectorSubcoreMesh(core_axis_name="c", subcore_axis_name="s",
                                  num_cores=NC, num_subcores=NS)
    out = pl.kernel(
        _sc_body,
        out_type=jax.ShapeDtypeStruct((NC, N_PAD, FH), jnp.float32),
        mesh=mesh,
        compiler_params=pltpu.CompilerParams(needs_layout_passes=False,
                                             use_tc_tiling_on_sc=False),
        scratch_types=[
            pltpu.VMEM_SHARED((N_PAD, FH), jnp.float32),
            pltpu.VMEM_SHARED((N_PAD, FH), jnp.float32),
            pltpu.VMEM_SHARED((N_PAD,), jnp.float32),
            pltpu.VMEM_SHARED((N_PAD,), jnp.float32),
            pltpu.VMEM((SUB,), jnp.int32),
            pltpu.VMEM((SUB,), jnp.int32),
            pltpu.VMEM((HSUB,), jnp.int32),
            pltpu.VMEM((HSUB,), jnp.int32),
            pltpu.VMEM((HSUB,), jnp.int32),
            pltpu.VMEM((HSUB,), jnp.int32),
            pltpu.VMEM((NPT,), jnp.int32),
            pltpu.VMEM((NPT,), jnp.float32),
            pltpu.VMEM((SUB, FH), jnp.float32),
            pltpu.VMEM((128, FH), jnp.float32),
            pltpu.VMEM((HSUB,), jnp.float32),
            pltpu.VMEM((FH,), jnp.float32),
            pltpu.SemaphoreType.DMA,
            pltpu.SemaphoreType.DMA,
            pltpu.SemaphoreType.DMA,
        ],
    )(feat_p, src_p, dst_p, emb2, bias)
    return jnp.concatenate([out[0, :N_NODES], out[1, :N_NODES]], axis=1)


# speedup vs baseline: 1.6462x; 1.1184x over previous
import functools

import jax
import jax.numpy as jnp
from jax import lax
from jax.experimental import pallas as pl
from jax.experimental.pallas import tpu as pltpu
from jax.experimental.pallas import tpu_sc as plsc

N_NODES = 10000
N_EDGES = 320000
IN_FEATS = 10000
OUT_FEATS = 128

NC = 2
NS = 16
L = 16
FH = OUT_FEATS // NC

NPT = 640
N_PAD = NS * NPT
NODE_CHUNKS = NPT // 128

SUB = 400
NSUB = 50
HSUB = 2000
HPAIRS = 5
EPT = NSUB * SUB


def _rsqrt_inplace(ref, n_vecs):
    def body(i, carry):
        x = jnp.maximum(ref[pl.ds(i * L, L)], 1.0)
        bits = lax.bitcast_convert_type(x, jnp.int32)
        y = lax.bitcast_convert_type(
            jnp.int32(0x5F3759DF) - lax.shift_right_arithmetic(bits, 1),
            jnp.float32)
        for _ in range(3):
            y = y * (1.5 - 0.5 * x * y * y)
        ref[pl.ds(i * L, L)] = y
        return carry

    lax.fori_loop(0, n_vecs, body, 0)


def _sc_body(feat_ref, src_ref, dst_ref, emb2_ref, bias_ref, out_ref,
             x_sp, accum, outdeg, indeg,
             ssub_a, dsub_a, ssub_b, dsub_b, hs_a, hd_a, hs_b, hd_b,
             featbuf, norm_v, rows_a, rows_b, rows_c, ones_v, biasv,
             gsem_a, gsem_b, hsem_a, hsem_b):
    c = lax.axis_index("c")
    s = lax.axis_index("s")
    base_n = s * NPT
    rows0 = rows_c

    zeros16 = jnp.zeros((L,), jnp.float32)

    def zero_rows(r, carry):
        for f in range(FH // L):
            rows0[r, pl.ds(f * L, L)] = zeros16
        return carry

    lax.fori_loop(0, 128, zero_rows, 0)

    def zero_norm(i, carry):
        norm_v[pl.ds(i * L, L)] = zeros16
        return carry

    lax.fori_loop(0, NPT // L, zero_norm, 0)

    ones16 = jnp.ones((L,), jnp.float32)

    def fill_ones(k, carry):
        ones_v[pl.ds(k * L, L)] = ones16
        return carry

    lax.fori_loop(0, HSUB // L, fill_ones, 0)

    for j in range(NODE_CHUNKS):
        pltpu.sync_copy(rows0, accum.at[pl.ds(base_n + j * 128, 128)])
    pltpu.sync_copy(norm_v, outdeg.at[pl.ds(base_n, NPT)])
    pltpu.sync_copy(norm_v, indeg.at[pl.ds(base_n, NPT)])

    pltpu.sync_copy(feat_ref.at[pl.ds(base_n, NPT)], featbuf)
    pltpu.sync_copy(bias_ref.at[pl.ds(c * FH, FH)], biasv)

    coff = (c * IN_FEATS).astype(jnp.int32)

    def add_off(i, carry):
        featbuf[pl.ds(i * L, L)] = featbuf[pl.ds(i * L, L)] + coff
        return carry

    lax.fori_loop(0, NPT // L, add_off, 0)

    plsc.subcore_barrier()

    def hist_pair(u, carry):
        pltpu.sync_copy(src_ref.at[s, pl.ds((2 * u) * HSUB, HSUB)], hs_a)
        pltpu.sync_copy(dst_ref.at[s, pl.ds((2 * u) * HSUB, HSUB)], hd_a)
        da = pltpu.async_copy(ones_v, outdeg.at[hs_a], hsem_a, add=True)
        db = pltpu.async_copy(ones_v, indeg.at[hd_a], hsem_b, add=True)
        pltpu.sync_copy(src_ref.at[s, pl.ds((2 * u + 1) * HSUB, HSUB)], hs_b)
        pltpu.sync_copy(dst_ref.at[s, pl.ds((2 * u + 1) * HSUB, HSUB)], hd_b)
        da.wait()
        db.wait()
        dc = pltpu.async_copy(ones_v, outdeg.at[hs_b], hsem_a, add=True)
        dd = pltpu.async_copy(ones_v, indeg.at[hd_b], hsem_b, add=True)
        dc.wait()
        dd.wait()
        return carry

    lax.fori_loop(0, HPAIRS, hist_pair, 0)
    plsc.subcore_barrier()

    pltpu.sync_copy(outdeg.at[pl.ds(base_n, NPT)], norm_v)
    _rsqrt_inplace(norm_v, NPT // L)

    lane_iota = lax.iota(jnp.int32, L)

    def scale_rows(j, bias_vecs=None):
        def group(g, carry2):
            nv16 = norm_v[pl.ds(j * 128 + g * L, L)]
            for r16 in range(L):
                bc = jnp.full((L,), jnp.sum(jnp.where(lane_iota == r16,
                                                      nv16, 0.0)))
                r = g * L + r16
                for f in range(FH // L):
                    v = rows0[r, pl.ds(f * L, L)] * bc
                    if bias_vecs is not None:
                        v = v + bias_vecs[f]
                    rows0[r, pl.ds(f * L, L)] = v
            return carry2

        lax.fori_loop(0, 128 // L, group, 0)

    def build_chunk(j, carry):
        pltpu.sync_copy(emb2_ref.at[featbuf.at[pl.ds(j * 128, 128)]], rows0)
        scale_rows(j)
        pltpu.sync_copy(rows0,
                        x_sp.at[pl.ds(c * N_PAD + base_n + j * 128, 128)])
        return carry

    lax.fori_loop(0, NODE_CHUNKS, build_chunk, 0)

    pltpu.sync_copy(indeg.at[pl.ds(base_n, NPT)], norm_v)
    _rsqrt_inplace(norm_v, NPT // L)
    plsc.subcore_barrier()

    xoff = c * N_PAD

    def add_xoff(buf):
        for i in range(SUB // L):
            buf[pl.ds(i * L, L)] = buf[pl.ds(i * L, L)] + xoff

    pltpu.sync_copy(src_ref.at[s, pl.ds(0, SUB)], ssub_a)
    add_xoff(ssub_a)
    pltpu.async_copy(x_sp.at[ssub_a], rows_a, gsem_a)

    def edge_pair(t, carry):
        pltpu.sync_copy(src_ref.at[s, pl.ds((2 * t + 1) * SUB, SUB)], ssub_b)
        add_xoff(ssub_b)
        db = pltpu.async_copy(x_sp.at[ssub_b], rows_b, gsem_b)
        pltpu.sync_copy(dst_ref.at[s, pl.ds(2 * t * SUB, SUB)], dsub_a)
        pltpu.make_async_copy(x_sp.at[ssub_a], rows_a, gsem_a).wait()
        pltpu.sync_copy(rows_a, accum.at[dsub_a], add=True)
        nxt = jnp.minimum(2 * t + 2, NSUB - 1)
        pltpu.sync_copy(src_ref.at[s, pl.ds(nxt * SUB, SUB)], ssub_a)
        add_xoff(ssub_a)
        pltpu.async_copy(x_sp.at[ssub_a], rows_a, gsem_a)
        pltpu.sync_copy(dst_ref.at[s, pl.ds((2 * t + 1) * SUB, SUB)], dsub_b)
        db.wait()
        pltpu.sync_copy(rows_b, accum.at[dsub_b], add=True)
        return carry

    lax.fori_loop(0, NSUB // 2, edge_pair, 0)
    pltpu.make_async_copy(x_sp.at[ssub_a], rows_a, gsem_a).wait()
    plsc.subcore_barrier()

    bias_vecs = [biasv[pl.ds(f * L, L)] for f in range(FH // L)]

    def out_chunk(j, carry):
        pltpu.sync_copy(accum.at[pl.ds(base_n + j * 128, 128)], rows0)
        scale_rows(j, bias_vecs)
        pltpu.sync_copy(rows0,
                        out_ref.at[c, pl.ds(base_n + j * 128, 128)])
        return carry

    lax.fori_loop(0, NODE_CHUNKS, out_chunk, 0)


@functools.partial(jax.jit, static_argnames=())
def kernel(feat, edge_index, emb, bias):
    feat = feat.astype(jnp.int32)
    src = edge_index[0].astype(jnp.int32)
    dst = edge_index[1].astype(jnp.int32)

    emb2 = jnp.concatenate([emb[:, :FH], emb[:, FH:]], axis=0)

    feat_p = jnp.concatenate(
        [feat, jnp.zeros((N_PAD - N_NODES,), jnp.int32)])
    src_p = src.reshape(NS, EPT)
    dst_p = dst.reshape(NS, EPT)

    mesh = plsc.VectorSubcoreMesh(core_axis_name="c", subcore_axis_name="s",
                                  num_cores=NC, num_subcores=NS)
    out = pl.kernel(
        _sc_body,
        out_type=jax.ShapeDtypeStruct((NC, N_PAD, FH), jnp.float32),
        mesh=mesh,
        compiler_params=pltpu.CompilerParams(needs_layout_passes=False,
                                             use_tc_tiling_on_sc=False),
        scratch_types=[
            pltpu.HBM((NC * N_PAD, FH), jnp.float32),
            pltpu.VMEM_SHARED((N_PAD, FH), jnp.float32),
            pltpu.VMEM_SHARED((N_PAD,), jnp.float32),
            pltpu.VMEM_SHARED((N_PAD,), jnp.float32),
            pltpu.VMEM((SUB,), jnp.int32),
            pltpu.VMEM((SUB,), jnp.int32),
            pltpu.VMEM((SUB,), jnp.int32),
            pltpu.VMEM((SUB,), jnp.int32),
            pltpu.VMEM((HSUB,), jnp.int32),
            pltpu.VMEM((HSUB,), jnp.int32),
            pltpu.VMEM((HSUB,), jnp.int32),
            pltpu.VMEM((HSUB,), jnp.int32),
            pltpu.VMEM((NPT,), jnp.int32),
            pltpu.VMEM((NPT,), jnp.float32),
            pltpu.VMEM((SUB, FH), jnp.float32),
            pltpu.VMEM((SUB, FH), jnp.float32),
            pltpu.VMEM((128, FH), jnp.float32),
            pltpu.VMEM((HSUB,), jnp.float32),
            pltpu.VMEM((FH,), jnp.float32),
            pltpu.SemaphoreType.DMA,
            pltpu.SemaphoreType.DMA,
            pltpu.SemaphoreType.DMA,
            pltpu.SemaphoreType.DMA,
        ],
    )(feat_p, src_p, dst_p, emb2, bias)
    return jnp.concatenate([out[0, :N_NODES], out[1, :N_NODES]], axis=1)


# speedup vs baseline: 1.6984x; 1.0317x over previous
import functools

import jax
import jax.numpy as jnp
from jax import lax
from jax.experimental import pallas as pl
from jax.experimental.pallas import tpu as pltpu
from jax.experimental.pallas import tpu_sc as plsc

N_NODES = 10000
N_EDGES = 320000
IN_FEATS = 10000
OUT_FEATS = 128

NC = 2
NS = 16
L = 16
FH = OUT_FEATS // NC

NPT = 640
N_PAD = NS * NPT
NODE_CHUNKS = NPT // 128

SUB = 400
NSUB = 50
HSUB = 2000
HPAIRS = 5
EPT = NSUB * SUB


def _rsqrt_inplace(ref, n_vecs):
    def body(i, carry):
        x = jnp.maximum(ref[pl.ds(i * L, L)], 1.0)
        bits = lax.bitcast_convert_type(x, jnp.int32)
        y = lax.bitcast_convert_type(
            jnp.int32(0x5F3759DF) - lax.shift_right_arithmetic(bits, 1),
            jnp.float32)
        for _ in range(3):
            y = y * (1.5 - 0.5 * x * y * y)
        ref[pl.ds(i * L, L)] = y
        return carry

    lax.fori_loop(0, n_vecs, body, 0)


def _sc_body(feat_ref, src_ref, dst_ref, emb2_ref, bias_ref, out_ref,
             x_sp, accum, outdeg, indeg,
             ssub_a, dsub_a, ssub_b, dsub_b, hs_a, hd_a, hs_b, hd_b,
             featbuf, norm_v, rows_a, rows_b, rows_c, rows_f, ones_v, biasv,
             gsem_a, gsem_b, hsem_a, hsem_b):
    c = lax.axis_index("c")
    s = lax.axis_index("s")
    base_n = s * NPT
    rows0 = rows_c

    zeros16 = jnp.zeros((L,), jnp.float32)

    def zero_rows(r, carry):
        for f in range(FH // L):
            rows0[r, pl.ds(f * L, L)] = zeros16
        return carry

    lax.fori_loop(0, 128, zero_rows, 0)

    def zero_norm(i, carry):
        norm_v[pl.ds(i * L, L)] = zeros16
        return carry

    lax.fori_loop(0, NPT // L, zero_norm, 0)

    ones16 = jnp.ones((L,), jnp.float32)

    def fill_ones(k, carry):
        ones_v[pl.ds(k * L, L)] = ones16
        return carry

    lax.fori_loop(0, HSUB // L, fill_ones, 0)

    for j in range(NODE_CHUNKS):
        pltpu.sync_copy(rows0, accum.at[pl.ds(base_n + j * 128, 128)])
    pltpu.sync_copy(norm_v, outdeg.at[pl.ds(base_n, NPT)])
    pltpu.sync_copy(norm_v, indeg.at[pl.ds(base_n, NPT)])

    pltpu.sync_copy(feat_ref.at[pl.ds(base_n, NPT)], featbuf)
    pltpu.sync_copy(bias_ref.at[pl.ds(c * FH, FH)], biasv)

    plsc.subcore_barrier()

    def hist_pair(u, carry):
        pltpu.sync_copy(src_ref.at[s, pl.ds((2 * u) * HSUB, HSUB)], hs_a)
        pltpu.sync_copy(dst_ref.at[s, pl.ds((2 * u) * HSUB, HSUB)], hd_a)
        da = pltpu.async_copy(ones_v, outdeg.at[hs_a], hsem_a, add=True)
        db = pltpu.async_copy(ones_v, indeg.at[hd_a], hsem_b, add=True)
        pltpu.sync_copy(src_ref.at[s, pl.ds((2 * u + 1) * HSUB, HSUB)], hs_b)
        pltpu.sync_copy(dst_ref.at[s, pl.ds((2 * u + 1) * HSUB, HSUB)], hd_b)
        da.wait()
        db.wait()
        dc = pltpu.async_copy(ones_v, outdeg.at[hs_b], hsem_a, add=True)
        dd = pltpu.async_copy(ones_v, indeg.at[hd_b], hsem_b, add=True)
        dc.wait()
        dd.wait()
        return carry

    lax.fori_loop(0, HPAIRS, hist_pair, 0)
    plsc.subcore_barrier()

    pltpu.sync_copy(outdeg.at[pl.ds(base_n, NPT)], norm_v)
    _rsqrt_inplace(norm_v, NPT // L)

    lane_iota = lax.iota(jnp.int32, L)

    def scale_rows(j, bias_vecs=None):
        def group(g, carry2):
            nv16 = norm_v[pl.ds(j * 128 + g * L, L)]
            for r16 in range(L):
                bc = jnp.full((L,), jnp.sum(jnp.where(lane_iota == r16,
                                                      nv16, 0.0)))
                r = g * L + r16
                for f in range(FH // L):
                    v = rows0[r, pl.ds(f * L, L)] * bc
                    if bias_vecs is not None:
                        v = v + bias_vecs[f]
                    rows0[r, pl.ds(f * L, L)] = v
            return carry2

        lax.fori_loop(0, 128 // L, group, 0)

    def build_chunk(j, carry):
        pltpu.sync_copy(emb2_ref.at[featbuf.at[pl.ds(j * 64, 64)]], rows_f)

        def bgroup(g, carry2):
            nv16 = norm_v[pl.ds(j * 64 + g * L, L)]
            for r16 in range(L):
                bc = jnp.full((L,), jnp.sum(jnp.where(lane_iota == r16,
                                                      nv16, 0.0)))
                r = g * L + r16
                for f in range(FH // L):
                    rows_c[r, pl.ds(f * L, L)] = (
                        rows_f[r, pl.ds(c * FH + f * L, L)] * bc)
            return carry2

        lax.fori_loop(0, 64 // L, bgroup, 0)
        pltpu.sync_copy(rows_c.at[pl.ds(0, 64)],
                        x_sp.at[pl.ds(c * N_PAD + base_n + j * 64, 64)])
        return carry

    lax.fori_loop(0, NPT // 64, build_chunk, 0)

    pltpu.sync_copy(indeg.at[pl.ds(base_n, NPT)], norm_v)
    _rsqrt_inplace(norm_v, NPT // L)
    plsc.subcore_barrier()

    xoff = c * N_PAD

    def add_xoff(buf):
        for i in range(SUB // L):
            buf[pl.ds(i * L, L)] = buf[pl.ds(i * L, L)] + xoff

    pltpu.sync_copy(src_ref.at[s, pl.ds(0, SUB)], ssub_a)
    add_xoff(ssub_a)
    pltpu.async_copy(x_sp.at[ssub_a], rows_a, gsem_a)

    def edge_pair(t, carry):
        pltpu.sync_copy(src_ref.at[s, pl.ds((2 * t + 1) * SUB, SUB)], ssub_b)
        add_xoff(ssub_b)
        db = pltpu.async_copy(x_sp.at[ssub_b], rows_b, gsem_b)
        pltpu.sync_copy(dst_ref.at[s, pl.ds(2 * t * SUB, SUB)], dsub_a)
        pltpu.make_async_copy(x_sp.at[ssub_a], rows_a, gsem_a).wait()
        pltpu.sync_copy(rows_a, accum.at[dsub_a], add=True)
        nxt = jnp.minimum(2 * t + 2, NSUB - 1)
        pltpu.sync_copy(src_ref.at[s, pl.ds(nxt * SUB, SUB)], ssub_a)
        add_xoff(ssub_a)
        pltpu.async_copy(x_sp.at[ssub_a], rows_a, gsem_a)
        pltpu.sync_copy(dst_ref.at[s, pl.ds((2 * t + 1) * SUB, SUB)], dsub_b)
        db.wait()
        pltpu.sync_copy(rows_b, accum.at[dsub_b], add=True)
        return carry

    lax.fori_loop(0, NSUB // 2, edge_pair, 0)
    pltpu.make_async_copy(x_sp.at[ssub_a], rows_a, gsem_a).wait()
    plsc.subcore_barrier()

    bias_vecs = [biasv[pl.ds(f * L, L)] for f in range(FH // L)]

    def out_chunk(j, carry):
        pltpu.sync_copy(accum.at[pl.ds(base_n + j * 128, 128)], rows0)
        scale_rows(j, bias_vecs)
        pltpu.sync_copy(rows0,
                        out_ref.at[pl.ds(base_n + j * 128, 128),
                                   pl.ds(c * FH, FH)])
        return carry

    lax.fori_loop(0, NODE_CHUNKS, out_chunk, 0)


@functools.partial(jax.jit, static_argnames=())
def kernel(feat, edge_index, emb, bias):
    feat = feat.astype(jnp.int32)
    src = edge_index[0].astype(jnp.int32)
    dst = edge_index[1].astype(jnp.int32)

    feat_p = jnp.concatenate(
        [feat, jnp.zeros((N_PAD - N_NODES,), jnp.int32)])
    src_p = src.reshape(NS, EPT)
    dst_p = dst.reshape(NS, EPT)

    mesh = plsc.VectorSubcoreMesh(core_axis_name="c", subcore_axis_name="s",
                                  num_cores=NC, num_subcores=NS)
    out = pl.kernel(
        _sc_body,
        out_type=jax.ShapeDtypeStruct((N_PAD, OUT_FEATS), jnp.float32),
        mesh=mesh,
        compiler_params=pltpu.CompilerParams(needs_layout_passes=False,
                                             use_tc_tiling_on_sc=False),
        scratch_types=[
            pltpu.HBM((NC * N_PAD, FH), jnp.float32),
            pltpu.VMEM_SHARED((N_PAD, FH), jnp.float32),
            pltpu.VMEM_SHARED((N_PAD,), jnp.float32),
            pltpu.VMEM_SHARED((N_PAD,), jnp.float32),
            pltpu.VMEM((SUB,), jnp.int32),
            pltpu.VMEM((SUB,), jnp.int32),
            pltpu.VMEM((SUB,), jnp.int32),
            pltpu.VMEM((SUB,), jnp.int32),
            pltpu.VMEM((HSUB,), jnp.int32),
            pltpu.VMEM((HSUB,), jnp.int32),
            pltpu.VMEM((HSUB,), jnp.int32),
            pltpu.VMEM((HSUB,), jnp.int32),
            pltpu.VMEM((NPT,), jnp.int32),
            pltpu.VMEM((NPT,), jnp.float32),
            pltpu.VMEM((SUB, FH), jnp.float32),
            pltpu.VMEM((SUB, FH), jnp.float32),
            pltpu.VMEM((128, FH), jnp.float32),
            pltpu.VMEM((64, OUT_FEATS), jnp.float32),
            pltpu.VMEM((HSUB,), jnp.float32),
            pltpu.VMEM((FH,), jnp.float32),
            pltpu.SemaphoreType.DMA,
            pltpu.SemaphoreType.DMA,
            pltpu.SemaphoreType.DMA,
            pltpu.SemaphoreType.DMA,
        ],
    )(feat_p, src_p, dst_p, emb, bias)
    return out[:N_NODES]
